# baseline (speedup 1.0000x reference)
import functools

import jax
import jax.numpy as jnp
from jax import lax
from jax.experimental import pallas as pl
from jax.experimental.pallas import tpu as pltpu
from jax.experimental.pallas import tpu_sc as plsc

N = 10000
D = 128
H = 16
OUT = 128
E = 320000

NC = 2
NS = 16
NW = NC * NS
CHUNK = 1024
CB = 10
EPAD = NW * CB * CHUNK
NPAD = 10112
TRASH = NPAD - N
RS_ACC = NPAD // NS

_mesh = plsc.VectorSubcoreMesh(core_axis_name="c", subcore_axis_name="s")


def _deg_body(er, out, dst_v, ones_v, tmp_v, deg_s):
    cid = lax.axis_index("c")
    sid = lax.axis_index("s")
    w = cid * NS + sid
    for j in range(RS_ACC // 16):
        tmp_v[pl.ds(j * 16, 16)] = jnp.zeros((16,), jnp.float32)
    pltpu.sync_copy(tmp_v, deg_s.at[pl.ds(sid * RS_ACC, RS_ACC)])
    pltpu.sync_copy(er.at[1, w], dst_v)
    for j in range(CHUNK // 16):
        ones_v[pl.ds(j * 16, 16)] = jnp.ones((16,), jnp.float32)
    plsc.subcore_barrier()

    def step(c, carry):
        pltpu.sync_copy(ones_v, deg_s.at[dst_v.at[c]], add=True)
        return carry

    lax.fori_loop(0, CB, step, 0)
    plsc.subcore_barrier()
    pltpu.sync_copy(deg_s.at[pl.ds(sid * RS_ACC, RS_ACC)], tmp_v)
    pltpu.sync_copy(tmp_v, out.at[pl.ds(cid * NPAD + sid * RS_ACC, RS_ACC)])


_deg = pl.kernel(
    _deg_body,
    out_type=jax.ShapeDtypeStruct((NC * NPAD,), jnp.float32),
    mesh=_mesh,
    scratch_types=[
        pltpu.VMEM((CB, CHUNK), jnp.int32),
        pltpu.VMEM((CHUNK,), jnp.float32),
        pltpu.VMEM((RS_ACC,), jnp.float32),
        pltpu.VMEM_SHARED((NPAD,), jnp.float32),
    ],
    compiler_params=pltpu.CompilerParams(use_tc_tiling_on_sc=False),
)


def _zero_acc(tmp_v, acc_s, sid):
    def zstep(j, carry):
        tmp_v[j] = jnp.zeros((16,), jnp.float32)
        return carry

    lax.fori_loop(0, RS_ACC, zstep, 0)
    pltpu.sync_copy(tmp_v, acc_s.at[pl.ds(sid * RS_ACC, RS_ACC), :])


def _hop_phase(er, out, src_v, dst_v, bufa_v, bufb_v, tmp_v, acc_s, tab_s,
               sema, semb, cid, sid, w):
    pltpu.sync_copy(tmp_v, tab_s.at[pl.ds(sid * RS_ACC, RS_ACC), :])
    pltpu.sync_copy(er.at[0, w], src_v)
    pltpu.sync_copy(er.at[1, w], dst_v)
    plsc.subcore_barrier()

    def gath(c, buf, sem):
        return pltpu.async_copy(tab_s.at[src_v.at[c]], buf, sem)

    def scat(c, buf):
        pltpu.sync_copy(buf, acc_s.at[dst_v.at[c]], add=True)

    gath(0, bufa_v, sema)

    def step(i, carry):
        c = 2 * i
        gath(c + 1, bufb_v, semb)
        pltpu.make_async_copy(tab_s.at[src_v.at[c]], bufa_v, sema).wait()
        scat(c, bufa_v)
        gath(c + 2, bufa_v, sema)
        pltpu.make_async_copy(tab_s.at[src_v.at[c]], bufb_v, semb).wait()
        scat(c + 1, bufb_v)
        return carry

    lax.fori_loop(0, CB // 2 - 1, step, 0)
    gath(CB - 1, bufb_v, semb)
    pltpu.make_async_copy(tab_s.at[src_v.at[0]], bufa_v, sema).wait()
    scat(CB - 2, bufa_v)
    pltpu.make_async_copy(tab_s.at[src_v.at[0]], bufb_v, semb).wait()
    scat(CB - 1, bufb_v)

    plsc.subcore_barrier()
    pltpu.sync_copy(acc_s.at[pl.ds(sid * RS_ACC, RS_ACC), :], tmp_v)
    pltpu.sync_copy(tmp_v, out.at[pl.ds(cid * NPAD + sid * RS_ACC, RS_ACC), :])


def _ld(hbm, buf_v, sid):
    pltpu.sync_copy(hbm.at[pl.ds(sid * RS_ACC, RS_ACC), :],
                    buf_v.at[pl.ds(0, RS_ACC), :])


def _hop1_body(h0, dis, er, out, src_v, dst_v, bufa_v, bufb_v, tmp_v, acc_s,
               tab_s, sema, semb):
    cid = lax.axis_index("c")
    sid = lax.axis_index("s")
    w = cid * NS + sid
    _zero_acc(tmp_v, acc_s, sid)
    _ld(h0, bufa_v, sid)
    _ld(dis, bufb_v, sid)

    def mstep(r, carry):
        tmp_v[r] = bufa_v[r] * bufb_v[r]
        return carry

    lax.fori_loop(0, RS_ACC, mstep, 0)
    _hop_phase(er, out, src_v, dst_v, bufa_v, bufb_v, tmp_v, acc_s, tab_s,
               sema, semb, cid, sid, w)


def _hop2_body(h0, dis, dis2, p, er, out, src_v, dst_v, bufa_v, bufb_v,
               tmp_v, acc_s, tab_s, sema, semb):
    cid = lax.axis_index("c")
    sid = lax.axis_index("s")
    w = cid * NS + sid
    _zero_acc(tmp_v, acc_s, sid)
    pltpu.sync_copy(p.at[pl.ds(sid * RS_ACC, RS_ACC), :],
                    bufa_v.at[pl.ds(0, RS_ACC), :])
    pltpu.sync_copy(p.at[pl.ds(NPAD + sid * RS_ACC, RS_ACC), :],
                    bufb_v.at[pl.ds(0, RS_ACC), :])

    def s1(r, carry):
        tmp_v[r] = bufa_v[r] + bufb_v[r]
        return carry

    lax.fori_loop(0, RS_ACC, s1, 0)
    _ld(h0, bufa_v, sid)
    _ld(dis, bufb_v, sid)

    def s2(r, carry):
        tmp_v[r] = tmp_v[r] + bufa_v[r] * bufb_v[r]
        return carry

    lax.fori_loop(0, RS_ACC, s2, 0)
    _ld(dis2, bufa_v, sid)

    def s3(r, carry):
        tmp_v[r] = tmp_v[r] * bufa_v[r]
        return carry

    lax.fori_loop(0, RS_ACC, s3, 0)
    _hop_phase(er, out, src_v, dst_v, bufa_v, bufb_v, tmp_v, acc_s, tab_s,
               sema, semb, cid, sid, w)


_hop_scratch = [
    pltpu.VMEM((CB, CHUNK), jnp.int32),
    pltpu.VMEM((CB, CHUNK), jnp.int32),
    pltpu.VMEM((CHUNK, H), jnp.float32),
    pltpu.VMEM((CHUNK, H), jnp.float32),
    pltpu.VMEM((RS_ACC, H), jnp.float32),
    pltpu.VMEM_SHARED((NPAD, H), jnp.float32),
    pltpu.VMEM_SHARED((NPAD, H), jnp.float32),
    pltpu.SemaphoreType.DMA,
    pltpu.SemaphoreType.DMA,
]

_hop1 = pl.kernel(
    _hop1_body,
    out_type=jax.ShapeDtypeStruct((NC * NPAD, H), jnp.float32),
    mesh=_mesh,
    scratch_types=list(_hop_scratch),
    compiler_params=pltpu.CompilerParams(use_tc_tiling_on_sc=False),
)

_hop2 = pl.kernel(
    _hop2_body,
    out_type=jax.ShapeDtypeStruct((NC * NPAD, H), jnp.float32),
    mesh=_mesh,
    scratch_types=list(_hop_scratch),
    compiler_params=pltpu.CompilerParams(use_tc_tiling_on_sc=False),
)


def _prep_body(x_ref, w_ref, degp_ref, h0_ref, dis_ref, dis2_ref):
    deg = degp_ref[0:N] + degp_ref[NPAD:NPAD + N] + 1.0
    dis1 = lax.rsqrt(deg)
    dis = jnp.broadcast_to(dis1.reshape(N, 1), (N, H))
    h0 = jnp.dot(x_ref[...], w_ref[...], preferred_element_type=jnp.float32)
    z = jnp.zeros((TRASH, H), jnp.float32)
    h0_ref[0:N, :] = h0
    h0_ref[N:NPAD, :] = z
    dis_ref[0:N, :] = dis
    dis_ref[N:NPAD, :] = z
    dis2_ref[0:N, :] = dis * dis
    dis2_ref[N:NPAD, :] = z


def _tc_prep(x, W_conv, degP):
    return pl.pallas_call(
        _prep_body,
        out_shape=[
            jax.ShapeDtypeStruct((NPAD, H), jnp.float32),
            jax.ShapeDtypeStruct((NPAD, H), jnp.float32),
            jax.ShapeDtypeStruct((NPAD, H), jnp.float32),
        ],
    )(x, W_conv, degP)


def _out_body(q_ref, p_ref, h0_ref, dis_ref, dis2_ref, bc_ref, wl_ref,
              bl_ref, out_ref):
    dis = dis_ref[0:N, :]
    g2 = dis2_ref[0:N, :] * (p_ref[0:N, :] + p_ref[NPAD:NPAD + N, :]
                             + dis * h0_ref[0:N, :])
    h2 = dis * (q_ref[0:N, :] + q_ref[NPAD:NPAD + N, :] + g2)
    a = jnp.maximum(h2 + bc_ref[...], 0.0)
    out_ref[...] = (jnp.dot(a, wl_ref[...], preferred_element_type=jnp.float32)
                    + bl_ref[...])


def _tc_out(Q, P, h0, dis, dis2, bc, wl, bl):
    return pl.pallas_call(
        _out_body,
        out_shape=jax.ShapeDtypeStruct((N, OUT), jnp.float32),
    )(Q, P, h0, dis, dis2, bc, wl, bl)


def kernel(x, edge_index, W_conv, b_conv, W_lin, b_lin):
    npad_e = EPAD - E
    pad_i = jnp.arange(npad_e, dtype=jnp.int32)
    pad = jnp.stack([(pad_i * 97) % N, N + (pad_i % TRASH)])
    er = jnp.concatenate([edge_index, pad], axis=1).reshape(2, NW, CB, CHUNK)

    degP = _deg(er)
    h0, dis, dis2 = _tc_prep(x, W_conv, degP)

    P = _hop1(h0, dis, er)
    Q = _hop2(h0, dis, dis2, P, er)
    out = _tc_out(Q, P, h0, dis, dis2,
                  b_conv.reshape(1, H), W_lin, b_lin.reshape(1, OUT))
    return out


# speedup vs baseline: 1.0923x; 1.0923x over previous
import functools

import jax
import jax.numpy as jnp
from jax import lax
from jax.experimental import pallas as pl
from jax.experimental.pallas import tpu as pltpu
from jax.experimental.pallas import tpu_sc as plsc

N = 10000
D = 128
H = 16
OUT = 128
E = 320000

NC = 2
NS = 16
NW = NC * NS
CHUNK = 1024
CB = 10
EPAD = NW * CB * CHUNK
NPAD = 10112
TRASH = NPAD - N
RS_ACC = NPAD // NS

_mesh = plsc.VectorSubcoreMesh(core_axis_name="c", subcore_axis_name="s")


def _deg_body(er, out, dst_v, ones_v, tmp_v, deg_s):
    cid = lax.axis_index("c")
    sid = lax.axis_index("s")
    w = cid * NS + sid
    for j in range(RS_ACC // 16):
        tmp_v[pl.ds(j * 16, 16)] = jnp.zeros((16,), jnp.float32)
    pltpu.sync_copy(tmp_v, deg_s.at[pl.ds(sid * RS_ACC, RS_ACC)])
    pltpu.sync_copy(er.at[1, w], dst_v)
    for j in range(CHUNK // 16):
        ones_v[pl.ds(j * 16, 16)] = jnp.ones((16,), jnp.float32)
    plsc.subcore_barrier()

    def step(c, carry):
        pltpu.sync_copy(ones_v, deg_s.at[dst_v.at[c]], add=True)
        return carry

    lax.fori_loop(0, CB, step, 0)
    plsc.subcore_barrier()
    pltpu.sync_copy(deg_s.at[pl.ds(sid * RS_ACC, RS_ACC)], tmp_v)
    pltpu.sync_copy(tmp_v, out.at[pl.ds(cid * NPAD + sid * RS_ACC, RS_ACC)])


_deg = pl.kernel(
    _deg_body,
    out_type=jax.ShapeDtypeStruct((NC * NPAD,), jnp.float32),
    mesh=_mesh,
    scratch_types=[
        pltpu.VMEM((CB, CHUNK), jnp.int32),
        pltpu.VMEM((CHUNK,), jnp.float32),
        pltpu.VMEM((RS_ACC,), jnp.float32),
        pltpu.VMEM_SHARED((NPAD,), jnp.float32),
    ],
    compiler_params=pltpu.CompilerParams(use_tc_tiling_on_sc=False),
)


UNROLL = 8


def _rows(body):
    def blk(b, carry):
        for k in range(UNROLL):
            body(b * UNROLL + k)
        return carry

    lax.fori_loop(0, RS_ACC // UNROLL, blk, 0)


def _zero_acc(tmp_v, acc_s, sid):
    def zstep(j):
        tmp_v[j] = jnp.zeros((16,), jnp.float32)

    _rows(zstep)
    pltpu.sync_copy(tmp_v, acc_s.at[pl.ds(sid * RS_ACC, RS_ACC), :])


def _hop_phase(er, out, src_v, dst_v, bufa_v, bufb_v, tmp_v, acc_s, tab_s,
               sema, semb, cid, sid, w):
    pltpu.sync_copy(tmp_v, tab_s.at[pl.ds(sid * RS_ACC, RS_ACC), :])
    pltpu.sync_copy(er.at[0, w], src_v)
    pltpu.sync_copy(er.at[1, w], dst_v)
    plsc.subcore_barrier()

    def gath(c, buf, sem):
        return pltpu.async_copy(tab_s.at[src_v.at[c]], buf, sem)

    def scat(c, buf):
        pltpu.sync_copy(buf, acc_s.at[dst_v.at[c]], add=True)

    gath(0, bufa_v, sema)

    def step(i, carry):
        c = 2 * i
        gath(c + 1, bufb_v, semb)
        pltpu.make_async_copy(tab_s.at[src_v.at[c]], bufa_v, sema).wait()
        scat(c, bufa_v)
        gath(c + 2, bufa_v, sema)
        pltpu.make_async_copy(tab_s.at[src_v.at[c]], bufb_v, semb).wait()
        scat(c + 1, bufb_v)
        return carry

    lax.fori_loop(0, CB // 2 - 1, step, 0)
    gath(CB - 1, bufb_v, semb)
    pltpu.make_async_copy(tab_s.at[src_v.at[0]], bufa_v, sema).wait()
    scat(CB - 2, bufa_v)
    pltpu.make_async_copy(tab_s.at[src_v.at[0]], bufb_v, semb).wait()
    scat(CB - 1, bufb_v)

    plsc.subcore_barrier()
    pltpu.sync_copy(acc_s.at[pl.ds(sid * RS_ACC, RS_ACC), :], tmp_v)
    pltpu.sync_copy(tmp_v, out.at[pl.ds(cid * NPAD + sid * RS_ACC, RS_ACC), :])


def _ld(hbm, buf_v, sid):
    pltpu.sync_copy(hbm.at[pl.ds(sid * RS_ACC, RS_ACC), :],
                    buf_v.at[pl.ds(0, RS_ACC), :])


def _hop1_body(h0, dis, er, out, src_v, dst_v, bufa_v, bufb_v, tmp_v, acc_s,
               tab_s, sema, semb):
    cid = lax.axis_index("c")
    sid = lax.axis_index("s")
    w = cid * NS + sid
    _zero_acc(tmp_v, acc_s, sid)
    _ld(h0, bufa_v, sid)
    _ld(dis, bufb_v, sid)

    def mstep(r):
        tmp_v[r] = bufa_v[r] * bufb_v[r]

    _rows(mstep)
    _hop_phase(er, out, src_v, dst_v, bufa_v, bufb_v, tmp_v, acc_s, tab_s,
               sema, semb, cid, sid, w)


def _hop2_body(h0, dis, dis2, p, er, out, src_v, dst_v, bufa_v, bufb_v,
               bufc_v, bufd_v, tmp_v, acc_s, tab_s, sema, semb):
    cid = lax.axis_index("c")
    sid = lax.axis_index("s")
    w = cid * NS + sid
    _zero_acc(tmp_v, acc_s, sid)
    pltpu.sync_copy(p.at[pl.ds(sid * RS_ACC, RS_ACC), :],
                    bufa_v.at[pl.ds(0, RS_ACC), :])
    pltpu.sync_copy(p.at[pl.ds(NPAD + sid * RS_ACC, RS_ACC), :],
                    bufb_v.at[pl.ds(0, RS_ACC), :])
    _ld(h0, bufc_v, sid)
    _ld(dis, bufd_v, sid)

    def s1(r):
        tmp_v[r] = bufa_v[r] + bufb_v[r] + bufc_v[r] * bufd_v[r]

    _rows(s1)
    _ld(dis2, bufa_v, sid)

    def s2(r):
        tmp_v[r] = tmp_v[r] * bufa_v[r]

    _rows(s2)
    _hop_phase(er, out, src_v, dst_v, bufa_v, bufb_v, tmp_v, acc_s, tab_s,
               sema, semb, cid, sid, w)


_hop_scratch = [
    pltpu.VMEM((CB, CHUNK), jnp.int32),
    pltpu.VMEM((CB, CHUNK), jnp.int32),
    pltpu.VMEM((CHUNK, H), jnp.float32),
    pltpu.VMEM((CHUNK, H), jnp.float32),
    pltpu.VMEM((RS_ACC, H), jnp.float32),
    pltpu.VMEM_SHARED((NPAD, H), jnp.float32),
    pltpu.VMEM_SHARED((NPAD, H), jnp.float32),
    pltpu.SemaphoreType.DMA,
    pltpu.SemaphoreType.DMA,
]

_hop1 = pl.kernel(
    _hop1_body,
    out_type=jax.ShapeDtypeStruct((NC * NPAD, H), jnp.float32),
    mesh=_mesh,
    scratch_types=list(_hop_scratch),
    compiler_params=pltpu.CompilerParams(use_tc_tiling_on_sc=False),
)

_hop2 = pl.kernel(
    _hop2_body,
    out_type=jax.ShapeDtypeStruct((NC * NPAD, H), jnp.float32),
    mesh=_mesh,
    scratch_types=(_hop_scratch[:4]
                   + [pltpu.VMEM((RS_ACC, H), jnp.float32),
                      pltpu.VMEM((RS_ACC, H), jnp.float32)]
                   + _hop_scratch[4:]),
    compiler_params=pltpu.CompilerParams(use_tc_tiling_on_sc=False),
)


def _prep_body(x_ref, w_ref, degp_ref, h0_ref, dis_ref, dis2_ref):
    deg = degp_ref[0:N] + degp_ref[NPAD:NPAD + N] + 1.0
    dis1 = lax.rsqrt(deg)
    dis = jnp.broadcast_to(dis1.reshape(N, 1), (N, H))
    h0 = jnp.dot(x_ref[...], w_ref[...], preferred_element_type=jnp.float32)
    z = jnp.zeros((TRASH, H), jnp.float32)
    h0_ref[0:N, :] = h0
    h0_ref[N:NPAD, :] = z
    dis_ref[0:N, :] = dis
    dis_ref[N:NPAD, :] = z
    dis2_ref[0:N, :] = dis * dis
    dis2_ref[N:NPAD, :] = z


def _tc_prep(x, W_conv, degP):
    return pl.pallas_call(
        _prep_body,
        out_shape=[
            jax.ShapeDtypeStruct((NPAD, H), jnp.float32),
            jax.ShapeDtypeStruct((NPAD, H), jnp.float32),
            jax.ShapeDtypeStruct((NPAD, H), jnp.float32),
        ],
    )(x, W_conv, degP)


def _out_body(q_ref, p_ref, h0_ref, dis_ref, dis2_ref, bc_ref, wl_ref,
              bl_ref, out_ref):
    dis = dis_ref[0:N, :]
    g2 = dis2_ref[0:N, :] * (p_ref[0:N, :] + p_ref[NPAD:NPAD + N, :]
                             + dis * h0_ref[0:N, :])
    h2 = dis * (q_ref[0:N, :] + q_ref[NPAD:NPAD + N, :] + g2)
    a = jnp.maximum(h2 + bc_ref[...], 0.0)
    out_ref[...] = (jnp.dot(a, wl_ref[...], preferred_element_type=jnp.float32)
                    + bl_ref[...])


def _tc_out(Q, P, h0, dis, dis2, bc, wl, bl):
    return pl.pallas_call(
        _out_body,
        out_shape=jax.ShapeDtypeStruct((N, OUT), jnp.float32),
    )(Q, P, h0, dis, dis2, bc, wl, bl)


def kernel(x, edge_index, W_conv, b_conv, W_lin, b_lin):
    npad_e = EPAD - E
    pad_i = jnp.arange(npad_e, dtype=jnp.int32)
    pad = jnp.stack([(pad_i * 97) % N, N + (pad_i % TRASH)])
    er = jnp.concatenate([edge_index, pad], axis=1).reshape(2, NW, CB, CHUNK)

    degP = _deg(er)
    h0, dis, dis2 = _tc_prep(x, W_conv, degP)

    P = _hop1(h0, dis, er)
    Q = _hop2(h0, dis, dis2, P, er)
    out = _tc_out(Q, P, h0, dis, dis2,
                  b_conv.reshape(1, H), W_lin, b_lin.reshape(1, OUT))
    return out


# speedup vs baseline: 1.1180x; 1.0236x over previous
import functools

import jax
import jax.numpy as jnp
from jax import lax
from jax.experimental import pallas as pl
from jax.experimental.pallas import tpu as pltpu
from jax.experimental.pallas import tpu_sc as plsc

N = 10000
D = 128
H = 16
OUT = 128
E = 320000

NC = 2
NS = 16
NW = NC * NS
CHUNK = 1024
CB = 10
EPAD = NW * CB * CHUNK
NPAD = 10112
TRASH = NPAD - N
RS_ACC = NPAD // NS

_mesh = plsc.VectorSubcoreMesh(core_axis_name="c", subcore_axis_name="s")


def _deg_body(er, out, dst_v, ones_v, tmp_v, deg_s):
    cid = lax.axis_index("c")
    sid = lax.axis_index("s")
    w = cid * NS + sid
    for j in range(RS_ACC // 16):
        tmp_v[pl.ds(j * 16, 16)] = jnp.zeros((16,), jnp.float32)
    pltpu.sync_copy(tmp_v, deg_s.at[pl.ds(sid * RS_ACC, RS_ACC)])
    pltpu.sync_copy(er.at[1, w], dst_v)
    for j in range(CHUNK // 16):
        ones_v[pl.ds(j * 16, 16)] = jnp.ones((16,), jnp.float32)
    plsc.subcore_barrier()

    def step(c, carry):
        pltpu.sync_copy(ones_v, deg_s.at[dst_v.at[c]], add=True)
        return carry

    lax.fori_loop(0, CB, step, 0)
    plsc.subcore_barrier()
    pltpu.sync_copy(deg_s.at[pl.ds(sid * RS_ACC, RS_ACC)], tmp_v)
    pltpu.sync_copy(tmp_v, out.at[pl.ds(cid * NPAD + sid * RS_ACC, RS_ACC)])


_deg = pl.kernel(
    _deg_body,
    out_type=jax.ShapeDtypeStruct((NC * NPAD,), jnp.float32),
    mesh=_mesh,
    scratch_types=[
        pltpu.VMEM((CB, CHUNK), jnp.int32),
        pltpu.VMEM((CHUNK,), jnp.float32),
        pltpu.VMEM((RS_ACC,), jnp.float32),
        pltpu.VMEM_SHARED((NPAD,), jnp.float32),
    ],
    compiler_params=pltpu.CompilerParams(use_tc_tiling_on_sc=False),
)


UNROLL = 8


def _rows(body):
    def blk(b, carry):
        for k in range(UNROLL):
            body(b * UNROLL + k)
        return carry

    lax.fori_loop(0, RS_ACC // UNROLL, blk, 0)


def _zero_acc(tmp_v, acc_s, sid):
    def zstep(j):
        tmp_v[j] = jnp.zeros((16,), jnp.float32)

    _rows(zstep)
    pltpu.sync_copy(tmp_v, acc_s.at[pl.ds(sid * RS_ACC, RS_ACC), :])


def _hop_phase(er, out, src_v, dst_v, bufa_v, bufb_v, tmp_v, acc_s, tab_s,
               sema, semb, cid, sid, w):
    pltpu.sync_copy(tmp_v, tab_s.at[pl.ds(sid * RS_ACC, RS_ACC), :])
    pltpu.sync_copy(er.at[0, w], src_v)
    pltpu.sync_copy(er.at[1, w], dst_v)
    plsc.subcore_barrier()

    def gath(c, buf, sem):
        return pltpu.async_copy(tab_s.at[src_v.at[c]], buf, sem)

    def scat(c, buf):
        pltpu.sync_copy(buf, acc_s.at[dst_v.at[c]], add=True)

    gath(0, bufa_v, sema)

    def step(i, carry):
        c = 2 * i
        gath(c + 1, bufb_v, semb)
        pltpu.make_async_copy(tab_s.at[src_v.at[c]], bufa_v, sema).wait()
        scat(c, bufa_v)
        gath(c + 2, bufa_v, sema)
        pltpu.make_async_copy(tab_s.at[src_v.at[c]], bufb_v, semb).wait()
        scat(c + 1, bufb_v)
        return carry

    lax.fori_loop(0, CB // 2 - 1, step, 0)
    gath(CB - 1, bufb_v, semb)
    pltpu.make_async_copy(tab_s.at[src_v.at[0]], bufa_v, sema).wait()
    scat(CB - 2, bufa_v)
    pltpu.make_async_copy(tab_s.at[src_v.at[0]], bufb_v, semb).wait()
    scat(CB - 1, bufb_v)

    plsc.subcore_barrier()
    pltpu.sync_copy(acc_s.at[pl.ds(sid * RS_ACC, RS_ACC), :], tmp_v)
    pltpu.sync_copy(tmp_v, out.at[pl.ds(cid * NPAD + sid * RS_ACC, RS_ACC), :])


def _ld(hbm, buf_v, sid):
    pltpu.sync_copy(hbm.at[pl.ds(sid * RS_ACC, RS_ACC), :],
                    buf_v.at[pl.ds(0, RS_ACC), :])


def _ld_async(hbm, buf_v, sid, sem):
    return pltpu.async_copy(hbm.at[pl.ds(sid * RS_ACC, RS_ACC), :],
                            buf_v.at[pl.ds(0, RS_ACC), :], sem)


def _ld_wait(hbm, buf_v, sid, sem):
    pltpu.make_async_copy(hbm.at[pl.ds(sid * RS_ACC, RS_ACC), :],
                          buf_v.at[pl.ds(0, RS_ACC), :], sem).wait()


def _hop1_body(h0, dis, er, out, src_v, dst_v, bufa_v, bufb_v, tmp_v, acc_s,
               tab_s, sema, semb):
    cid = lax.axis_index("c")
    sid = lax.axis_index("s")
    w = cid * NS + sid
    _ld_async(h0, bufa_v, sid, sema)
    _ld_async(dis, bufb_v, sid, semb)
    _zero_acc(tmp_v, acc_s, sid)
    _ld_wait(h0, bufa_v, sid, sema)
    _ld_wait(dis, bufb_v, sid, semb)

    def mstep(r):
        tmp_v[r] = bufa_v[r] * bufb_v[r]

    _rows(mstep)
    _hop_phase(er, out, src_v, dst_v, bufa_v, bufb_v, tmp_v, acc_s, tab_s,
               sema, semb, cid, sid, w)


def _hop2_body(h0, dis, dis2, p, er, out, src_v, dst_v, bufa_v, bufb_v,
               bufc_v, bufd_v, tmp_v, acc_s, tab_s, sema, semb):
    cid = lax.axis_index("c")
    sid = lax.axis_index("s")
    w = cid * NS + sid
    pltpu.async_copy(p.at[pl.ds(sid * RS_ACC, RS_ACC), :],
                     bufa_v.at[pl.ds(0, RS_ACC), :], sema)
    pltpu.async_copy(p.at[pl.ds(NPAD + sid * RS_ACC, RS_ACC), :],
                     bufb_v.at[pl.ds(0, RS_ACC), :], semb)
    _zero_acc(tmp_v, acc_s, sid)
    pltpu.make_async_copy(p.at[pl.ds(sid * RS_ACC, RS_ACC), :],
                          bufa_v.at[pl.ds(0, RS_ACC), :], sema).wait()
    pltpu.make_async_copy(p.at[pl.ds(NPAD + sid * RS_ACC, RS_ACC), :],
                          bufb_v.at[pl.ds(0, RS_ACC), :], semb).wait()
    _ld(h0, bufc_v, sid)
    _ld(dis, bufd_v, sid)

    def s1(r):
        tmp_v[r] = bufa_v[r] + bufb_v[r] + bufc_v[r] * bufd_v[r]

    _rows(s1)
    _ld(dis2, bufa_v, sid)

    def s2(r):
        tmp_v[r] = tmp_v[r] * bufa_v[r]

    _rows(s2)
    _hop_phase(er, out, src_v, dst_v, bufa_v, bufb_v, tmp_v, acc_s, tab_s,
               sema, semb, cid, sid, w)


_hop_scratch = [
    pltpu.VMEM((CB, CHUNK), jnp.int32),
    pltpu.VMEM((CB, CHUNK), jnp.int32),
    pltpu.VMEM((CHUNK, H), jnp.float32),
    pltpu.VMEM((CHUNK, H), jnp.float32),
    pltpu.VMEM((RS_ACC, H), jnp.float32),
    pltpu.VMEM_SHARED((NPAD, H), jnp.float32),
    pltpu.VMEM_SHARED((NPAD, H), jnp.float32),
    pltpu.SemaphoreType.DMA,
    pltpu.SemaphoreType.DMA,
]

_hop1 = pl.kernel(
    _hop1_body,
    out_type=jax.ShapeDtypeStruct((NC * NPAD, H), jnp.float32),
    mesh=_mesh,
    scratch_types=list(_hop_scratch),
    compiler_params=pltpu.CompilerParams(use_tc_tiling_on_sc=False),
)

_hop2 = pl.kernel(
    _hop2_body,
    out_type=jax.ShapeDtypeStruct((NC * NPAD, H), jnp.float32),
    mesh=_mesh,
    scratch_types=(_hop_scratch[:4]
                   + [pltpu.VMEM((RS_ACC, H), jnp.float32),
                      pltpu.VMEM((RS_ACC, H), jnp.float32)]
                   + _hop_scratch[4:]),
    compiler_params=pltpu.CompilerParams(use_tc_tiling_on_sc=False),
)


def _prep_body(x_ref, w_ref, degp_ref, h0_ref, dis_ref, dis2_ref):
    deg = degp_ref[0:N] + degp_ref[NPAD:NPAD + N] + 1.0
    dis1 = lax.rsqrt(deg)
    dis = jnp.broadcast_to(dis1.reshape(N, 1), (N, H))
    h0 = jnp.dot(x_ref[...], w_ref[...], preferred_element_type=jnp.float32)
    z = jnp.zeros((TRASH, H), jnp.float32)
    h0_ref[0:N, :] = h0
    h0_ref[N:NPAD, :] = z
    dis_ref[0:N, :] = dis
    dis_ref[N:NPAD, :] = z
    dis2_ref[0:N, :] = dis * dis
    dis2_ref[N:NPAD, :] = z


def _tc_prep(x, W_conv, degP):
    return pl.pallas_call(
        _prep_body,
        out_shape=[
            jax.ShapeDtypeStruct((NPAD, H), jnp.float32),
            jax.ShapeDtypeStruct((NPAD, H), jnp.float32),
            jax.ShapeDtypeStruct((NPAD, H), jnp.float32),
        ],
    )(x, W_conv, degP)


def _out_body(q_ref, p_ref, h0_ref, dis_ref, dis2_ref, bc_ref, wl_ref,
              bl_ref, out_ref):
    dis = dis_ref[0:N, :]
    g2 = dis2_ref[0:N, :] * (p_ref[0:N, :] + p_ref[NPAD:NPAD + N, :]
                             + dis * h0_ref[0:N, :])
    h2 = dis * (q_ref[0:N, :] + q_ref[NPAD:NPAD + N, :] + g2)
    a = jnp.maximum(h2 + bc_ref[...], 0.0)
    out_ref[...] = (jnp.dot(a, wl_ref[...], preferred_element_type=jnp.float32)
                    + bl_ref[...])


def _tc_out(Q, P, h0, dis, dis2, bc, wl, bl):
    return pl.pallas_call(
        _out_body,
        out_shape=jax.ShapeDtypeStruct((N, OUT), jnp.float32),
    )(Q, P, h0, dis, dis2, bc, wl, bl)


def kernel(x, edge_index, W_conv, b_conv, W_lin, b_lin):
    npad_e = EPAD - E
    pad_i = jnp.arange(npad_e, dtype=jnp.int32)
    pad = jnp.stack([(pad_i * 97) % N, N + (pad_i % TRASH)])
    er = jnp.concatenate([edge_index, pad], axis=1).reshape(2, NW, CB, CHUNK)

    degP = _deg(er)
    h0, dis, dis2 = _tc_prep(x, W_conv, degP)

    P = _hop1(h0, dis, er)
    Q = _hop2(h0, dis, dis2, P, er)
    out = _tc_out(Q, P, h0, dis, dis2,
                  b_conv.reshape(1, H), W_lin, b_lin.reshape(1, OUT))
    return out
